# baseline (speedup 1.0000x reference)
import functools

import jax
import jax.numpy as jnp
from jax import lax
from jax.experimental import pallas as pl
from jax.experimental.pallas import tpu as pltpu
from jax.experimental.pallas import tpu_sc as plsc

N = 10000
D = 128
G = 100
E = 320000
STRIDE = N // G

NC = 2
NS = 16
NW = NC * NS
CHUNK = 128
IB = 4
CPT = 80
NGRP = CPT // IB
EPW = CPT * CHUNK
E_PAD = EPW * NW
FAST_CID = 1
FAST_CPT = 144
SLOW_CPT = 16
CPT_MAX = 144
N_PAD = N + 112
ZR = N_PAD // NS

RB = 1000

_ROW_CHUNKS = [(0, 128), (128, 128), (256, 128), (384, 128), (512, 120)]


def _sc_mesh():
    return plsc.VectorSubcoreMesh(
        core_axis_name="c", subcore_axis_name="s", num_cores=NC, num_subcores=NS
    )


def _sc_body(h_hbm, src_hbm, dst_hbm, zagg_hbm, agg_out,
             src_v0, dst_v0, src_v1, dst_v1, buf0, buf1, agg_s, sem0, sem1):
    cid = lax.axis_index("c")
    sid = lax.axis_index("s")
    wid = sid * NC + cid
    bufs = (buf0, buf1)
    sems = (sem0, sem1)

    pltpu.sync_copy(zagg_hbm.at[pl.ds(0, CHUNK)], buf0)
    for (off, sz) in _ROW_CHUNKS:
        base = sid * ZR + off
        pltpu.sync_copy(buf0.at[pl.ds(0, sz)], agg_s.at[pl.ds(base, sz)])
    plsc.subcore_barrier()

    def stage(g, sv, dv):
        pltpu.sync_copy(src_hbm.at[wid, pl.ds(g * IB, IB)], sv)
        pltpu.sync_copy(dst_hbm.at[wid, pl.ds(g * IB, IB)], dv)

    def start(sv, j, p):
        pltpu.async_copy(h_hbm.at[sv.at[j]], bufs[p], sems[p])

    def wait_scatter(dv, j, p):
        pltpu.make_async_copy(h_hbm.at[pl.ds(0, CHUNK)], bufs[p], sems[p]).wait()
        pltpu.sync_copy(bufs[p], agg_s.at[dv.at[j]], add=True)

    stage(0, src_v0, dst_v0)
    start(src_v0, 0, 0)

    T = jnp.where(cid == FAST_CID, FAST_CPT // (2 * IB), SLOW_CPT // (2 * IB))

    def body2(t, carry):
        stage(2 * t + 1, src_v1, dst_v1)
        for j in range(IB):
            p = j % 2
            if j < IB - 1:
                start(src_v0, j + 1, 1 - p)
            else:
                start(src_v1, 0, 1 - p)
            wait_scatter(dst_v0, j, p)

        @pl.when(t < T - 1)
        def _():
            stage(2 * t + 2, src_v0, dst_v0)

        for j in range(IB):
            p = j % 2
            if j < IB - 1:
                start(src_v1, j + 1, 1 - p)
            else:
                @pl.when(t < T - 1)
                def _():
                    start(src_v0, 0, 1 - p)
            wait_scatter(dst_v1, j, p)
        return carry

    lax.fori_loop(0, T, body2, 0)
    plsc.subcore_barrier()

    for (off, sz) in _ROW_CHUNKS:
        base = sid * ZR + off
        pltpu.sync_copy(agg_s.at[pl.ds(base, sz)], buf0.at[pl.ds(0, sz)])
        pltpu.sync_copy(buf0.at[pl.ds(0, sz)], agg_out.at[cid, pl.ds(base, sz)])


def _deg_body(dst_hbm, zagg_hbm, ones_hbm, deg_out,
              dst_v, buf, deg_s):
    cid = lax.axis_index("c")
    sid = lax.axis_index("s")
    wid = sid * NC + cid

    pltpu.sync_copy(zagg_hbm.at[pl.ds(0, CHUNK)], buf)
    for (off, sz) in _ROW_CHUNKS:
        base = sid * ZR + off
        pltpu.sync_copy(buf.at[pl.ds(0, sz)], deg_s.at[pl.ds(base, sz)])
    pltpu.sync_copy(ones_hbm, buf)
    plsc.subcore_barrier()

    def group(g, carry):
        pltpu.sync_copy(dst_hbm.at[wid, pl.ds(g * IB, IB)], dst_v)
        for j in range(IB):
            pltpu.sync_copy(buf, deg_s.at[dst_v.at[j]], add=True)
        return carry

    lax.fori_loop(0, NGRP, group, 0)
    plsc.subcore_barrier()

    for (off, sz) in _ROW_CHUNKS:
        base = sid * ZR + off
        pltpu.sync_copy(deg_s.at[pl.ds(base, sz)], buf.at[pl.ds(0, sz)])
        pltpu.sync_copy(buf.at[pl.ds(0, sz)], deg_out.at[cid, pl.ds(base, sz)])


def _make_sc():
    return pl.kernel(
        _sc_body,
        out_type=jax.ShapeDtypeStruct((NC, N_PAD, D), jnp.float32),
        mesh=_sc_mesh(),
        scratch_types=[
            pltpu.VMEM((IB, CHUNK), jnp.int32),
            pltpu.VMEM((IB, CHUNK), jnp.int32),
            pltpu.VMEM((IB, CHUNK), jnp.int32),
            pltpu.VMEM((IB, CHUNK), jnp.int32),
            pltpu.VMEM((CHUNK, D), jnp.float32),
            pltpu.VMEM((CHUNK, D), jnp.float32),
            pltpu.VMEM_SHARED((N_PAD, D), jnp.float32),
            pltpu.SemaphoreType.DMA,
            pltpu.SemaphoreType.DMA,
        ],
    )


def _make_deg():
    return pl.kernel(
        _deg_body,
        out_type=jax.ShapeDtypeStruct((NC, N_PAD, D), jnp.float32),
        mesh=_sc_mesh(),
        scratch_types=[
            pltpu.VMEM((IB, CHUNK), jnp.int32),
            pltpu.VMEM((CHUNK, D), jnp.float32),
            pltpu.VMEM_SHARED((N_PAD, D), jnp.float32),
        ],
    )


def _proj_body(x_ref, whT, wsT, bs, o_ref):
    xb = x_ref[...]
    hh = jnp.dot(xb, whT[...], preferred_element_type=jnp.float32)
    hs = jnp.dot(xb, wsT[...], preferred_element_type=jnp.float32) + bs[...]
    rid = lax.broadcasted_iota(jnp.int32, (RB, 1), 0)
    h = jnp.where(rid % STRIDE == 0, hh, hs)
    o_ref[...] = jnp.maximum(h, 0.0)


_proj = pl.pallas_call(
    _proj_body,
    grid=(N // RB,),
    in_specs=[
        pl.BlockSpec((RB, D), lambda i: (i, 0)),
        pl.BlockSpec((D, D), lambda i: (0, 0)),
        pl.BlockSpec((D, D), lambda i: (0, 0)),
        pl.BlockSpec((1, D), lambda i: (0, 0)),
    ],
    out_specs=pl.BlockSpec((RB, D), lambda i: (i, 0)),
    out_shape=jax.ShapeDtypeStruct((N, D), jnp.float32),
)


def _layer_body(last, p_ref, dp_ref, r_ref, wlT, bl, wrT, *outs):
    p = p_ref[...]
    dp = dp_ref[...]
    deg = dp[0, :, 0:1] + dp[1, :, 0:1]
    inv = 1.0 / jnp.maximum(deg, 1.0)
    agg = (p[0] + p[1]) * inv
    h = (jnp.dot(agg, wlT[...], preferred_element_type=jnp.float32) + bl[...]
         + jnp.dot(r_ref[...], wrT[...], preferred_element_type=jnp.float32))
    if last:
        outs[0][...] = h
        outs[1][...] = jnp.concatenate(
            [h[k * STRIDE:k * STRIDE + 1] for k in range(RB // STRIDE)],
            axis=0)[None]
    else:
        outs[0][...] = jnp.maximum(h, 0.0)


def _make_layer(last):
    out_shape = [jax.ShapeDtypeStruct((N, D), jnp.float32)]
    out_specs = [pl.BlockSpec((RB, D), lambda i: (i, 0))]
    if last:
        out_shape.append(
            jax.ShapeDtypeStruct((N // RB, RB // STRIDE, D), jnp.float32))
        out_specs.append(
            pl.BlockSpec((1, RB // STRIDE, D), lambda i: (i, 0, 0)))
    return pl.pallas_call(
        functools.partial(_layer_body, last),
        grid=(N // RB,),
        in_specs=[
            pl.BlockSpec((NC, RB, D), lambda i: (0, i, 0)),
            pl.BlockSpec((NC, RB, D), lambda i: (0, i, 0)),
            pl.BlockSpec((RB, D), lambda i: (i, 0)),
            pl.BlockSpec((D, D), lambda i: (0, 0)),
            pl.BlockSpec((1, D), lambda i: (0, 0)),
            pl.BlockSpec((D, D), lambda i: (0, 0)),
        ],
        out_specs=out_specs if last else out_specs[0],
        out_shape=out_shape if last else out_shape[0],
    )


def _head_body(hyd_ref, w1T, b1, a_ref, w2T, b2, o_ref):
    z = jnp.dot(hyd_ref[...], w1T[...], preferred_element_type=jnp.float32) + b1[...]
    z = jnp.maximum(z, 0.0) + a_ref[0, 0] * jnp.minimum(z, 0.0)
    o_ref[...] = jnp.dot(z, w2T[...], preferred_element_type=jnp.float32) + b2[...]


_head = pl.pallas_call(
    _head_body,
    out_shape=jax.ShapeDtypeStruct((G, 1), jnp.float32),
)


def kernel(x, edge_index, ptr, W_hydro, W_ship, b_ship, Wl, bl, Wr, W1, b1,
           prelu_a, W2, b2):
    f32 = jnp.float32
    src = edge_index[0]
    dst = edge_index[1]
    npad = E_PAD - E
    pad_src = jnp.arange(npad, dtype=jnp.int32) % N
    pad_dst = N + (jnp.arange(npad, dtype=jnp.int32) % (N_PAD - N))
    src_p = jnp.concatenate([src, pad_src])
    dst_p = jnp.concatenate([dst, pad_dst])
    dst_r = dst_p.reshape(NW, CPT, CHUNK)

    def asym(a):
        nfast = NS * FAST_CPT * CHUNK
        fast = a[:nfast].reshape(NS, FAST_CPT, CHUNK)
        slow = a[nfast:].reshape(NS, SLOW_CPT, CHUNK)
        slow = jnp.concatenate(
            [slow, jnp.zeros((NS, CPT_MAX - SLOW_CPT, CHUNK), jnp.int32)], axis=1)
        pair = [None, None]
        pair[FAST_CID] = fast
        pair[1 - FAST_CID] = slow
        return jnp.stack(pair, axis=1).reshape(NW, CPT_MAX, CHUNK)

    src_r = asym(src_p)
    dst_a = asym(dst_p)
    zagg = jnp.zeros((CHUNK, D), f32)
    ones128 = jnp.ones((CHUNK, D), f32)

    scn = _make_sc()
    sc_deg = _make_deg()
    layer_mid = _make_layer(False)
    layer_last = _make_layer(True)

    r = _proj(x, W_hydro.T, W_ship.T, b_ship.reshape(1, D))
    deg_p = sc_deg(dst_r, zagg, ones128)

    agg_p = scn(r, src_r, dst_a, zagg)
    r = layer_mid(agg_p, deg_p, r, Wl[0].T, bl[0].reshape(1, D), Wr[0].T)
    agg_p = scn(r, src_r, dst_a, zagg)
    r = layer_mid(agg_p, deg_p, r, Wl[1].T, bl[1].reshape(1, D), Wr[1].T)
    agg_p = scn(r, src_r, dst_a, zagg)
    _, hyd = layer_last(agg_p, deg_p, r, Wl[2].T, bl[2].reshape(1, D), Wr[2].T)
    hyd = hyd.reshape(G, D)

    return _head(hyd, W1.T, b1.reshape(1, -1), prelu_a.reshape(1, 1),
                 W2.T, b2.reshape(1, 1))


# speedup vs baseline: 1.0915x; 1.0915x over previous
import functools

import jax
import jax.numpy as jnp
from jax import lax
from jax.experimental import pallas as pl
from jax.experimental.pallas import tpu as pltpu
from jax.experimental.pallas import tpu_sc as plsc

N = 10000
D = 128
G = 100
E = 320000
STRIDE = N // G

NC = 2
NS = 16
NW = NC * NS
CHUNK = 128
IB = 4
CPT = 80
NGRP = CPT // IB
EPW = CPT * CHUNK
E_PAD = EPW * NW
FAST_CID = 1
FAST_CPT = 128
SLOW_CPT = 32
CPT_MAX = 128
N_PAD = N + 112
ZR = N_PAD // NS

RB = 1000

_ROW_CHUNKS = [(0, 128), (128, 128), (256, 128), (384, 128), (512, 120)]


def _sc_mesh():
    return plsc.VectorSubcoreMesh(
        core_axis_name="c", subcore_axis_name="s", num_cores=NC, num_subcores=NS
    )


def _sc_body(h_hbm, src_hbm, dst_hbm, zagg_hbm, agg_out,
             src_v0, dst_v0, src_v1, dst_v1, buf0, buf1, agg_s,
             sem0, sem1, sem2, sem3):
    cid = lax.axis_index("c")
    sid = lax.axis_index("s")
    wid = sid * NC + cid
    bufs = (buf0, buf1)
    gsems = (sem0, sem1)
    ssems = (sem2, sem3)

    pltpu.sync_copy(zagg_hbm.at[pl.ds(0, CHUNK)], buf0)
    for (off, sz) in _ROW_CHUNKS:
        base = sid * ZR + off
        pltpu.sync_copy(buf0.at[pl.ds(0, sz)], agg_s.at[pl.ds(base, sz)])
    plsc.subcore_barrier()

    def stage(g, sv, dv):
        pltpu.sync_copy(src_hbm.at[wid, pl.ds(g * IB, IB)], sv)
        pltpu.sync_copy(dst_hbm.at[wid, pl.ds(g * IB, IB)], dv)

    def start(sv, j, p):
        pltpu.async_copy(h_hbm.at[sv.at[j]], bufs[p], gsems[p])

    def wait_gather(p):
        pltpu.make_async_copy(h_hbm.at[pl.ds(0, CHUNK)], bufs[p], gsems[p]).wait()

    def start_scatter(dv, j, p):
        pltpu.async_copy(bufs[p], agg_s.at[dv.at[j]], ssems[p], add=True)

    def wait_scatter(p):
        pltpu.make_async_copy(bufs[p], agg_s.at[pl.ds(0, CHUNK)], ssems[p]).wait()

    stage(0, src_v0, dst_v0)
    start(src_v0, 0, 0)

    T = jnp.where(cid == FAST_CID, FAST_CPT // (2 * IB), SLOW_CPT // (2 * IB))

    def body2(t, carry):
        for j in range(IB):
            p = j % 2
            if j == 0:
                @pl.when(t > 0)
                def _():
                    wait_scatter(1 - p)
                stage(2 * t + 1, src_v1, dst_v1)
            else:
                wait_scatter(1 - p)
            if j < IB - 1:
                start(src_v0, j + 1, 1 - p)
            else:
                start(src_v1, 0, 1 - p)
            wait_gather(p)
            start_scatter(dst_v0, j, p)

        for j in range(IB):
            p = j % 2
            wait_scatter(1 - p)
            if j == 0:
                @pl.when(t < T - 1)
                def _():
                    stage(2 * t + 2, src_v0, dst_v0)
            if j < IB - 1:
                start(src_v1, j + 1, 1 - p)
            else:
                @pl.when(t < T - 1)
                def _():
                    start(src_v0, 0, 1 - p)
            wait_gather(p)
            start_scatter(dst_v1, j, p)
        return carry

    lax.fori_loop(0, T, body2, 0)
    wait_scatter(1)
    plsc.subcore_barrier()

    for (off, sz) in _ROW_CHUNKS:
        base = sid * ZR + off
        pltpu.sync_copy(agg_s.at[pl.ds(base, sz)], buf0.at[pl.ds(0, sz)])
        pltpu.sync_copy(buf0.at[pl.ds(0, sz)], agg_out.at[cid, pl.ds(base, sz)])


def _deg_body(dst_hbm, zagg_hbm, ones_hbm, deg_out,
              dst_v, buf, deg_s):
    cid = lax.axis_index("c")
    sid = lax.axis_index("s")
    wid = sid * NC + cid

    pltpu.sync_copy(zagg_hbm.at[pl.ds(0, CHUNK)], buf)
    for (off, sz) in _ROW_CHUNKS:
        base = sid * ZR + off
        pltpu.sync_copy(buf.at[pl.ds(0, sz)], deg_s.at[pl.ds(base, sz)])
    pltpu.sync_copy(ones_hbm, buf)
    plsc.subcore_barrier()

    def group(g, carry):
        pltpu.sync_copy(dst_hbm.at[wid, pl.ds(g * IB, IB)], dst_v)
        for j in range(IB):
            pltpu.sync_copy(buf, deg_s.at[dst_v.at[j]], add=True)
        return carry

    lax.fori_loop(0, NGRP, group, 0)
    plsc.subcore_barrier()

    for (off, sz) in _ROW_CHUNKS:
        base = sid * ZR + off
        pltpu.sync_copy(deg_s.at[pl.ds(base, sz)], buf.at[pl.ds(0, sz)])
        pltpu.sync_copy(buf.at[pl.ds(0, sz)], deg_out.at[cid, pl.ds(base, sz)])


def _make_sc():
    return pl.kernel(
        _sc_body,
        out_type=jax.ShapeDtypeStruct((NC, N_PAD, D), jnp.float32),
        mesh=_sc_mesh(),
        scratch_types=[
            pltpu.VMEM((IB, CHUNK), jnp.int32),
            pltpu.VMEM((IB, CHUNK), jnp.int32),
            pltpu.VMEM((IB, CHUNK), jnp.int32),
            pltpu.VMEM((IB, CHUNK), jnp.int32),
            pltpu.VMEM((CHUNK, D), jnp.float32),
            pltpu.VMEM((CHUNK, D), jnp.float32),
            pltpu.VMEM_SHARED((N_PAD, D), jnp.float32),
            pltpu.SemaphoreType.DMA,
            pltpu.SemaphoreType.DMA,
            pltpu.SemaphoreType.DMA,
            pltpu.SemaphoreType.DMA,
        ],
    )


def _make_deg():
    return pl.kernel(
        _deg_body,
        out_type=jax.ShapeDtypeStruct((NC, N_PAD, D), jnp.float32),
        mesh=_sc_mesh(),
        scratch_types=[
            pltpu.VMEM((IB, CHUNK), jnp.int32),
            pltpu.VMEM((CHUNK, D), jnp.float32),
            pltpu.VMEM_SHARED((N_PAD, D), jnp.float32),
        ],
    )


def _proj_body(x_ref, whT, wsT, bs, o_ref):
    xb = x_ref[...]
    hh = jnp.dot(xb, whT[...], preferred_element_type=jnp.float32)
    hs = jnp.dot(xb, wsT[...], preferred_element_type=jnp.float32) + bs[...]
    rid = lax.broadcasted_iota(jnp.int32, (RB, 1), 0)
    h = jnp.where(rid % STRIDE == 0, hh, hs)
    o_ref[...] = jnp.maximum(h, 0.0)


_proj = pl.pallas_call(
    _proj_body,
    grid=(N // RB,),
    in_specs=[
        pl.BlockSpec((RB, D), lambda i: (i, 0)),
        pl.BlockSpec((D, D), lambda i: (0, 0)),
        pl.BlockSpec((D, D), lambda i: (0, 0)),
        pl.BlockSpec((1, D), lambda i: (0, 0)),
    ],
    out_specs=pl.BlockSpec((RB, D), lambda i: (i, 0)),
    out_shape=jax.ShapeDtypeStruct((N, D), jnp.float32),
)


def _layer_body(last, p_ref, dp_ref, r_ref, wlT, bl, wrT, *outs):
    p = p_ref[...]
    dp = dp_ref[...]
    deg = dp[0, :, 0:1] + dp[1, :, 0:1]
    inv = 1.0 / jnp.maximum(deg, 1.0)
    agg = (p[0] + p[1]) * inv
    h = (jnp.dot(agg, wlT[...], preferred_element_type=jnp.float32) + bl[...]
         + jnp.dot(r_ref[...], wrT[...], preferred_element_type=jnp.float32))
    if last:
        outs[0][...] = h
        outs[1][...] = jnp.concatenate(
            [h[k * STRIDE:k * STRIDE + 1] for k in range(RB // STRIDE)],
            axis=0)[None]
    else:
        outs[0][...] = jnp.maximum(h, 0.0)


def _make_layer(last):
    out_shape = [jax.ShapeDtypeStruct((N, D), jnp.float32)]
    out_specs = [pl.BlockSpec((RB, D), lambda i: (i, 0))]
    if last:
        out_shape.append(
            jax.ShapeDtypeStruct((N // RB, RB // STRIDE, D), jnp.float32))
        out_specs.append(
            pl.BlockSpec((1, RB // STRIDE, D), lambda i: (i, 0, 0)))
    return pl.pallas_call(
        functools.partial(_layer_body, last),
        grid=(N // RB,),
        in_specs=[
            pl.BlockSpec((NC, RB, D), lambda i: (0, i, 0)),
            pl.BlockSpec((NC, RB, D), lambda i: (0, i, 0)),
            pl.BlockSpec((RB, D), lambda i: (i, 0)),
            pl.BlockSpec((D, D), lambda i: (0, 0)),
            pl.BlockSpec((1, D), lambda i: (0, 0)),
            pl.BlockSpec((D, D), lambda i: (0, 0)),
        ],
        out_specs=out_specs if last else out_specs[0],
        out_shape=out_shape if last else out_shape[0],
    )


def _head_body(hyd_ref, w1T, b1, a_ref, w2T, b2, o_ref):
    z = jnp.dot(hyd_ref[...], w1T[...], preferred_element_type=jnp.float32) + b1[...]
    z = jnp.maximum(z, 0.0) + a_ref[0, 0] * jnp.minimum(z, 0.0)
    o_ref[...] = jnp.dot(z, w2T[...], preferred_element_type=jnp.float32) + b2[...]


_head = pl.pallas_call(
    _head_body,
    out_shape=jax.ShapeDtypeStruct((G, 1), jnp.float32),
)


def kernel(x, edge_index, ptr, W_hydro, W_ship, b_ship, Wl, bl, Wr, W1, b1,
           prelu_a, W2, b2):
    f32 = jnp.float32
    src = edge_index[0]
    dst = edge_index[1]
    npad = E_PAD - E
    pad_src = jnp.arange(npad, dtype=jnp.int32) % N
    pad_dst = N + (jnp.arange(npad, dtype=jnp.int32) % (N_PAD - N))
    src_p = jnp.concatenate([src, pad_src])
    dst_p = jnp.concatenate([dst, pad_dst])
    dst_r = dst_p.reshape(NW, CPT, CHUNK)

    def asym(a):
        nfast = NS * FAST_CPT * CHUNK
        fast = a[:nfast].reshape(NS, FAST_CPT, CHUNK)
        slow = a[nfast:].reshape(NS, SLOW_CPT, CHUNK)
        slow = jnp.concatenate(
            [slow, jnp.zeros((NS, CPT_MAX - SLOW_CPT, CHUNK), jnp.int32)], axis=1)
        pair = [None, None]
        pair[FAST_CID] = fast
        pair[1 - FAST_CID] = slow
        return jnp.stack(pair, axis=1).reshape(NW, CPT_MAX, CHUNK)

    src_r = asym(src_p)
    dst_a = asym(dst_p)
    zagg = jnp.zeros((CHUNK, D), f32)
    ones128 = jnp.ones((CHUNK, D), f32)

    scn = _make_sc()
    sc_deg = _make_deg()
    layer_mid = _make_layer(False)
    layer_last = _make_layer(True)

    r = _proj(x, W_hydro.T, W_ship.T, b_ship.reshape(1, D))
    deg_p = sc_deg(dst_r, zagg, ones128)

    agg_p = scn(r, src_r, dst_a, zagg)
    r = layer_mid(agg_p, deg_p, r, Wl[0].T, bl[0].reshape(1, D), Wr[0].T)
    agg_p = scn(r, src_r, dst_a, zagg)
    r = layer_mid(agg_p, deg_p, r, Wl[1].T, bl[1].reshape(1, D), Wr[1].T)
    agg_p = scn(r, src_r, dst_a, zagg)
    _, hyd = layer_last(agg_p, deg_p, r, Wl[2].T, bl[2].reshape(1, D), Wr[2].T)
    hyd = hyd.reshape(G, D)

    return _head(hyd, W1.T, b1.reshape(1, -1), prelu_a.reshape(1, 1),
                 W2.T, b2.reshape(1, 1))


# speedup vs baseline: 1.2390x; 1.1352x over previous
import functools

import jax
import jax.numpy as jnp
from jax import lax
from jax.experimental import pallas as pl
from jax.experimental.pallas import tpu as pltpu
from jax.experimental.pallas import tpu_sc as plsc

N = 10000
D = 128
G = 100
E = 320000
STRIDE = N // G

NC = 2
NS = 16
NW = NC * NS
CHUNK = 128
IB = 8
CPT = 80
NGRP = CPT // IB
EPW = CPT * CHUNK
E_PAD = EPW * NW
FAST_CID = 1
FAST_CPT = 128
SLOW_CPT = 32
CPT_MAX = 128
N_PAD = N + 112
ZR = N_PAD // NS

RB = 1000

_ROW_CHUNKS = [(0, 128), (128, 128), (256, 128), (384, 128), (512, 120)]


def _sc_mesh():
    return plsc.VectorSubcoreMesh(
        core_axis_name="c", subcore_axis_name="s", num_cores=NC, num_subcores=NS
    )


def _sc_body(h_hbm, src_hbm, dst_hbm, zagg_hbm, agg_out,
             src_v0, dst_v0, src_v1, dst_v1, buf0, buf1, agg_s,
             sem0, sem1, sem2, sem3, stsem):
    cid = lax.axis_index("c")
    sid = lax.axis_index("s")
    wid = sid * NC + cid
    bufs = (buf0, buf1)
    gsems = (sem0, sem1)
    ssems = (sem2, sem3)

    pltpu.sync_copy(zagg_hbm.at[pl.ds(0, CHUNK)], buf0)
    for (off, sz) in _ROW_CHUNKS:
        base = sid * ZR + off
        pltpu.sync_copy(buf0.at[pl.ds(0, sz)], agg_s.at[pl.ds(base, sz)])
    plsc.subcore_barrier()

    def stage(g, sv, dv):
        pltpu.async_copy(src_hbm.at[wid, pl.ds(g * IB, IB)], sv, stsem)
        pltpu.async_copy(dst_hbm.at[wid, pl.ds(g * IB, IB)], dv, stsem)

    def wait_stage(sv, dv):
        pltpu.make_async_copy(src_hbm.at[wid, pl.ds(0, IB)], sv, stsem).wait()
        pltpu.make_async_copy(dst_hbm.at[wid, pl.ds(0, IB)], dv, stsem).wait()

    def start(sv, j, p):
        pltpu.async_copy(h_hbm.at[sv.at[j]], bufs[p], gsems[p])

    def wait_gather(p):
        pltpu.make_async_copy(h_hbm.at[pl.ds(0, CHUNK)], bufs[p], gsems[p]).wait()

    def start_scatter(dv, j, p):
        pltpu.async_copy(bufs[p], agg_s.at[dv.at[j]], ssems[p], add=True)

    def wait_scatter(p):
        pltpu.make_async_copy(bufs[p], agg_s.at[pl.ds(0, CHUNK)], ssems[p]).wait()

    stage(0, src_v0, dst_v0)
    wait_stage(src_v0, dst_v0)
    start(src_v0, 0, 0)

    T = jnp.where(cid == FAST_CID, FAST_CPT // (2 * IB), SLOW_CPT // (2 * IB))

    def body2(t, carry):
        for j in range(IB):
            p = j % 2
            if j == 0:
                @pl.when(t > 0)
                def _():
                    wait_scatter(1 - p)
                stage(2 * t + 1, src_v1, dst_v1)
            else:
                wait_scatter(1 - p)
            if j < IB - 1:
                start(src_v0, j + 1, 1 - p)
            else:
                wait_stage(src_v1, dst_v1)
                start(src_v1, 0, 1 - p)
            wait_gather(p)
            start_scatter(dst_v0, j, p)

        for j in range(IB):
            p = j % 2
            wait_scatter(1 - p)
            if j == 0:
                @pl.when(t < T - 1)
                def _():
                    stage(2 * t + 2, src_v0, dst_v0)
            if j < IB - 1:
                start(src_v1, j + 1, 1 - p)
            else:
                @pl.when(t < T - 1)
                def _():
                    wait_stage(src_v0, dst_v0)
                    start(src_v0, 0, 1 - p)
            wait_gather(p)
            start_scatter(dst_v1, j, p)
        return carry

    lax.fori_loop(0, T, body2, 0)
    wait_scatter(1)
    plsc.subcore_barrier()

    for (off, sz) in _ROW_CHUNKS:
        base = sid * ZR + off
        pltpu.sync_copy(agg_s.at[pl.ds(base, sz)], buf0.at[pl.ds(0, sz)])
        pltpu.sync_copy(buf0.at[pl.ds(0, sz)], agg_out.at[cid, pl.ds(base, sz)])


def _deg_body(dst_hbm, zagg_hbm, ones_hbm, deg_out,
              dst_v, buf, deg_s, dsem):
    cid = lax.axis_index("c")
    sid = lax.axis_index("s")
    wid = sid * NC + cid

    pltpu.sync_copy(zagg_hbm.at[pl.ds(0, CHUNK)], buf)
    for (off, sz) in _ROW_CHUNKS:
        base = sid * ZR + off
        pltpu.sync_copy(buf.at[pl.ds(0, sz)], deg_s.at[pl.ds(base, sz)])
    pltpu.sync_copy(ones_hbm, buf)
    plsc.subcore_barrier()

    def group(g, carry):
        pltpu.sync_copy(dst_hbm.at[wid, pl.ds(g * IB, IB)], dst_v)
        for j in range(IB):
            pltpu.async_copy(buf, deg_s.at[dst_v.at[j]], dsem, add=True)
        for j in range(IB):
            pltpu.make_async_copy(buf, deg_s.at[pl.ds(0, CHUNK)], dsem).wait()
        return carry

    lax.fori_loop(0, NGRP, group, 0)
    plsc.subcore_barrier()

    for (off, sz) in _ROW_CHUNKS:
        base = sid * ZR + off
        pltpu.sync_copy(deg_s.at[pl.ds(base, sz)], buf.at[pl.ds(0, sz)])
        pltpu.sync_copy(buf.at[pl.ds(0, sz)], deg_out.at[cid, pl.ds(base, sz)])


def _make_sc():
    return pl.kernel(
        _sc_body,
        out_type=jax.ShapeDtypeStruct((NC, N_PAD, D), jnp.float32),
        mesh=_sc_mesh(),
        scratch_types=[
            pltpu.VMEM((IB, CHUNK), jnp.int32),
            pltpu.VMEM((IB, CHUNK), jnp.int32),
            pltpu.VMEM((IB, CHUNK), jnp.int32),
            pltpu.VMEM((IB, CHUNK), jnp.int32),
            pltpu.VMEM((CHUNK, D), jnp.float32),
            pltpu.VMEM((CHUNK, D), jnp.float32),
            pltpu.VMEM_SHARED((N_PAD, D), jnp.float32),
            pltpu.SemaphoreType.DMA,
            pltpu.SemaphoreType.DMA,
            pltpu.SemaphoreType.DMA,
            pltpu.SemaphoreType.DMA,
            pltpu.SemaphoreType.DMA,
        ],
    )


def _make_deg():
    return pl.kernel(
        _deg_body,
        out_type=jax.ShapeDtypeStruct((NC, N_PAD, D), jnp.float32),
        mesh=_sc_mesh(),
        scratch_types=[
            pltpu.VMEM((IB, CHUNK), jnp.int32),
            pltpu.VMEM((CHUNK, D), jnp.float32),
            pltpu.VMEM_SHARED((N_PAD, D), jnp.float32),
            pltpu.SemaphoreType.DMA,
        ],
    )


def _proj_body(x_ref, whT, wsT, bs, o_ref):
    xb = x_ref[...]
    hh = jnp.dot(xb, whT[...], preferred_element_type=jnp.float32)
    hs = jnp.dot(xb, wsT[...], preferred_element_type=jnp.float32) + bs[...]
    rid = lax.broadcasted_iota(jnp.int32, (RB, 1), 0)
    h = jnp.where(rid % STRIDE == 0, hh, hs)
    o_ref[...] = jnp.maximum(h, 0.0)


_proj = pl.pallas_call(
    _proj_body,
    grid=(N // RB,),
    in_specs=[
        pl.BlockSpec((RB, D), lambda i: (i, 0)),
        pl.BlockSpec((D, D), lambda i: (0, 0)),
        pl.BlockSpec((D, D), lambda i: (0, 0)),
        pl.BlockSpec((1, D), lambda i: (0, 0)),
    ],
    out_specs=pl.BlockSpec((RB, D), lambda i: (i, 0)),
    out_shape=jax.ShapeDtypeStruct((N, D), jnp.float32),
)


def _layer_body(last, p_ref, dp_ref, r_ref, wlT, bl, wrT, *outs):
    p = p_ref[...]
    dp = dp_ref[...]
    deg = dp[0, :, 0:1] + dp[1, :, 0:1]
    inv = 1.0 / jnp.maximum(deg, 1.0)
    agg = (p[0] + p[1]) * inv
    h = (jnp.dot(agg, wlT[...], preferred_element_type=jnp.float32) + bl[...]
         + jnp.dot(r_ref[...], wrT[...], preferred_element_type=jnp.float32))
    if last:
        outs[0][...] = h
        outs[1][...] = jnp.concatenate(
            [h[k * STRIDE:k * STRIDE + 1] for k in range(RB // STRIDE)],
            axis=0)[None]
    else:
        outs[0][...] = jnp.maximum(h, 0.0)


def _make_layer(last):
    out_shape = [jax.ShapeDtypeStruct((N, D), jnp.float32)]
    out_specs = [pl.BlockSpec((RB, D), lambda i: (i, 0))]
    if last:
        out_shape.append(
            jax.ShapeDtypeStruct((N // RB, RB // STRIDE, D), jnp.float32))
        out_specs.append(
            pl.BlockSpec((1, RB // STRIDE, D), lambda i: (i, 0, 0)))
    return pl.pallas_call(
        functools.partial(_layer_body, last),
        grid=(N // RB,),
        in_specs=[
            pl.BlockSpec((NC, RB, D), lambda i: (0, i, 0)),
            pl.BlockSpec((NC, RB, D), lambda i: (0, i, 0)),
            pl.BlockSpec((RB, D), lambda i: (i, 0)),
            pl.BlockSpec((D, D), lambda i: (0, 0)),
            pl.BlockSpec((1, D), lambda i: (0, 0)),
            pl.BlockSpec((D, D), lambda i: (0, 0)),
        ],
        out_specs=out_specs if last else out_specs[0],
        out_shape=out_shape if last else out_shape[0],
    )


def _head_body(hyd_ref, w1T, b1, a_ref, w2T, b2, o_ref):
    z = jnp.dot(hyd_ref[...], w1T[...], preferred_element_type=jnp.float32) + b1[...]
    z = jnp.maximum(z, 0.0) + a_ref[0, 0] * jnp.minimum(z, 0.0)
    o_ref[...] = jnp.dot(z, w2T[...], preferred_element_type=jnp.float32) + b2[...]


_head = pl.pallas_call(
    _head_body,
    out_shape=jax.ShapeDtypeStruct((G, 1), jnp.float32),
)


def kernel(x, edge_index, ptr, W_hydro, W_ship, b_ship, Wl, bl, Wr, W1, b1,
           prelu_a, W2, b2):
    f32 = jnp.float32
    src = edge_index[0]
    dst = edge_index[1]
    npad = E_PAD - E
    pad_src = jnp.arange(npad, dtype=jnp.int32) % N
    pad_dst = N + (jnp.arange(npad, dtype=jnp.int32) % (N_PAD - N))
    src_p = jnp.concatenate([src, pad_src])
    dst_p = jnp.concatenate([dst, pad_dst])
    dst_r = dst_p.reshape(NW, CPT, CHUNK)

    def asym(a):
        nfast = NS * FAST_CPT * CHUNK
        fast = a[:nfast].reshape(NS, FAST_CPT, CHUNK)
        slow = a[nfast:].reshape(NS, SLOW_CPT, CHUNK)
        slow = jnp.concatenate(
            [slow, jnp.zeros((NS, CPT_MAX - SLOW_CPT, CHUNK), jnp.int32)], axis=1)
        pair = [None, None]
        pair[FAST_CID] = fast
        pair[1 - FAST_CID] = slow
        return jnp.stack(pair, axis=1).reshape(NW, CPT_MAX, CHUNK)

    src_r = asym(src_p)
    dst_a = asym(dst_p)
    zagg = jnp.zeros((CHUNK, D), f32)
    ones128 = jnp.ones((CHUNK, D), f32)

    scn = _make_sc()
    sc_deg = _make_deg()
    layer_mid = _make_layer(False)
    layer_last = _make_layer(True)

    r = _proj(x, W_hydro.T, W_ship.T, b_ship.reshape(1, D))
    deg_p = sc_deg(dst_r, zagg, ones128)

    agg_p = scn(r, src_r, dst_a, zagg)
    r = layer_mid(agg_p, deg_p, r, Wl[0].T, bl[0].reshape(1, D), Wr[0].T)
    agg_p = scn(r, src_r, dst_a, zagg)
    r = layer_mid(agg_p, deg_p, r, Wl[1].T, bl[1].reshape(1, D), Wr[1].T)
    agg_p = scn(r, src_r, dst_a, zagg)
    _, hyd = layer_last(agg_p, deg_p, r, Wl[2].T, bl[2].reshape(1, D), Wr[2].T)
    hyd = hyd.reshape(G, D)

    return _head(hyd, W1.T, b1.reshape(1, -1), prelu_a.reshape(1, 1),
                 W2.T, b2.reshape(1, 1))
